# baseline (speedup 1.0000x reference)
import jax
import jax.numpy as jnp
from jax import lax
from jax.experimental import pallas as pl
from jax.experimental.pallas import tpu as pltpu
from jax.experimental.pallas import tpu_sc as plsc

N = 10000
E = 320000
F_IN = 128
HID = 128
OUT = 128
G = 128

NS = 16
L = 16

NPAD = 10240
NT = NPAD // NS
RPT = 160
EROWS = NS * RPT
CSL = NPAD * G // NS
ZB = 2048
CR = 32
CHUNKS = [(q * CR, CR) for q in range(RPT // CR)]


def _rsqrt16(x):
    i = plsc.bitcast(x, jnp.int32)
    y = plsc.bitcast(jnp.int32(0x5F3759DF) - (i >> 1), jnp.float32)
    for _ in range(3):
        y = y * (1.5 - 0.5 * x * y * y)
    return y


def _sc_body(srce_hbm, dste_hbm, batch_hbm, d_out, dinv_out, cnt_out,
             combo_v, src_c, dst_c, acc_s, bat_s, zbuf_v,
             idx_c, val_c,
             D_sh, deg_sh, combo_sh, cnt_sh,
             sem_stage, sem_scat, sem_zero, sem_bat):
    sid = lax.axis_index("s")

    zero16f = jnp.zeros((L,), jnp.float32)
    one16f = jnp.ones((L,), jnp.float32)
    iota16 = lax.broadcasted_iota(jnp.int32, (L,), 0)
    rbase = sid * RPT
    sb = sid * NT

    stage_bat = pltpu.async_copy(batch_hbm.at[pl.ds(sb, NT)], bat_s,
                                 sem_bat)

    def zz(i, c):
        zbuf_v[pl.ds(i * L, L)] = zero16f
        return c
    lax.fori_loop(0, ZB // L, zz, 0)
    zdeg_d = pltpu.async_copy(zbuf_v.at[pl.ds(0, NT)],
                              deg_sh.at[pl.ds(sb, NT)], sem_zero)
    zdescs = [pltpu.async_copy(zbuf_v,
                               D_sh.at[pl.ds(sid * CSL + q * ZB, ZB)],
                               sem_zero)
              for q in range(CSL // ZB)]
    zdescs.append(pltpu.async_copy(zbuf_v.at[pl.ds(0, G)], cnt_sh,
                                   sem_zero))

    for p in range(2):
        def fill1(i, c, p=p):
            val_c[p, i // 8, pl.ds((i % 8) * L, L)] = one16f
            return c
        lax.fori_loop(0, (CR * 128) // L, fill1, 0)
    zdeg_d.wait()
    plsc.subcore_barrier()

    stage_d = {0: pltpu.async_copy(
        dste_hbm.at[pl.ds(rbase, CR)], idx_c.at[0], sem_stage)}
    scat_d = {}
    for k, (off, rk) in enumerate(CHUNKS):
        p = k % 2
        stage_d.pop(k).wait()
        scat_d[k] = [pltpu.async_copy(val_c.at[p, j],
                                      deg_sh.at[idx_c.at[p, j]],
                                      sem_scat, add=True)
                     for j in range(rk)]
        if k - 1 in scat_d:
            for d in scat_d.pop(k - 1):
                d.wait()
        if k + 1 < len(CHUNKS):
            stage_d[k + 1] = pltpu.async_copy(
                dste_hbm.at[pl.ds(rbase + CHUNKS[k + 1][0], CR)],
                idx_c.at[1 - p], sem_stage)
    for d in scat_d.pop(len(CHUNKS) - 1):
        d.wait()

    plsc.subcore_barrier()

    pltpu.sync_copy(deg_sh.at[pl.ds(sb, NT)], acc_s)
    stage_bat.wait()

    def dinv_calc(q, c):
        dv = _rsqrt16(acc_s[pl.ds(q * L, L)] + 1.0)
        acc_s[pl.ds(q * L, L)] = dv
        combo_v[pl.ds(q * L, L)] = (
            (plsc.bitcast(dv, jnp.int32) & jnp.int32(-128))
            | bat_s[pl.ds(q * L, L)])
        return c
    lax.fori_loop(0, NT // L, dinv_calc, 0)
    pltpu.sync_copy(combo_v.at[pl.ds(0, NT)], combo_sh.at[pl.ds(sb, NT)])
    pltpu.sync_copy(acc_s, dinv_out.at[pl.ds(sb, NT)])

    for d in zdescs:
        d.wait()
    plsc.subcore_barrier()
    combo_d = pltpu.async_copy(combo_sh, combo_v, sem_bat)

    stage_d = {0: [pltpu.async_copy(
        srce_hbm.at[pl.ds(rbase, CR)], src_c.at[0], sem_stage),
        pltpu.async_copy(
        dste_hbm.at[pl.ds(rbase, CR)], dst_c.at[0], sem_stage)]}
    scat_d = {}
    for k, (off, rk) in enumerate(CHUNKS):
        p = k % 2
        if k == 0:
            combo_d.wait()
        for d in stage_d.pop(k):
            d.wait()
        if k - 2 in scat_d:
            for d in scat_d.pop(k - 2):
                d.wait()

        def coeff(i, c, p=p):
            r = i // 8
            cl = (i % 8) * L
            s16 = src_c[p, r, pl.ds(cl, L)]
            d16 = dst_c[p, r, pl.ds(cl, L)]
            cb = plsc.load_gather(combo_v, [d16])
            idx_c[p, r, pl.ds(cl, L)] = s16 * G + (cb & jnp.int32(127))
            val_c[p, r, pl.ds(cl, L)] = plsc.bitcast(
                cb & jnp.int32(-128), jnp.float32)
            return c
        lax.fori_loop(0, (rk * 128) // L, coeff, 0)
        scat_d[k] = [pltpu.async_copy(val_c.at[p, j],
                                      D_sh.at[idx_c.at[p, j]],
                                      sem_scat, add=True)
                     for j in range(rk)]
        if k + 1 < len(CHUNKS):
            off2 = CHUNKS[k + 1][0]
            stage_d[k + 1] = [pltpu.async_copy(
                srce_hbm.at[pl.ds(rbase + off2, CR)],
                src_c.at[1 - p], sem_stage),
                pltpu.async_copy(
                dste_hbm.at[pl.ds(rbase + off2, CR)],
                dst_c.at[1 - p], sem_stage)]
    for k in list(scat_d):
        for d in scat_d.pop(k):
            d.wait()

    for q in range(NT // L):
        k = q * L
        nvec = sb + k + iota16
        dv = acc_s[pl.ds(k, L)]
        g16 = bat_s[pl.ds(k, L)]
        valid = nvec < N
        idx_c[0, k // 128, pl.ds(k % 128, L)] = jnp.where(
            valid, nvec * G + g16, 0)
        val_c[0, k // 128, pl.ds(k % 128, L)] = jnp.where(valid, dv, 0.0)
    sl_d = [pltpu.async_copy(val_c.at[0, j], D_sh.at[idx_c.at[0, j]],
                             sem_scat, add=True)
            for j in range(NT // 128)]

    for q in range(NT // L):
        k = q * L
        nvec = sb + k + iota16
        valid = nvec < N
        idx_c[1, k // 128, pl.ds(k % 128, L)] = jnp.where(
            valid, bat_s[pl.ds(k, L)], 0)
        val_c[1, k // 128, pl.ds(k % 128, L)] = jnp.where(valid, 1.0, 0.0)
    sl_d += [pltpu.async_copy(val_c.at[1, j], cnt_sh.at[idx_c.at[1, j]],
                              sem_scat, add=True)
             for j in range(NT // 128)]
    for d in sl_d:
        d.wait()

    plsc.subcore_barrier()

    pltpu.sync_copy(D_sh.at[pl.ds(sid * CSL, CSL)],
                    d_out.at[pl.ds(sid * CSL, CSL)])

    @pl.when(sid == 0)
    def _():
        pltpu.sync_copy(cnt_sh, cnt_out)


@jax.jit
def _sc_coeffs(srce2d, dste2d, batch_pad):
    mesh = plsc.VectorSubcoreMesh(core_axis_name="c", subcore_axis_name="s",
                                  num_cores=1, num_subcores=NS)
    f = pl.kernel(
        _sc_body,
        out_type=(jax.ShapeDtypeStruct((NPAD * G,), jnp.float32),
                  jax.ShapeDtypeStruct((NPAD,), jnp.float32),
                  jax.ShapeDtypeStruct((G,), jnp.float32)),
        mesh=mesh,
        compiler_params=pltpu.CompilerParams(needs_layout_passes=False),
        scratch_types=[
            pltpu.VMEM((NPAD,), jnp.int32),
            pltpu.VMEM((2, CR, 128), jnp.int32),
            pltpu.VMEM((2, CR, 128), jnp.int32),
            pltpu.VMEM((NT,), jnp.float32),
            pltpu.VMEM((NT,), jnp.int32),
            pltpu.VMEM((ZB,), jnp.float32),
            pltpu.VMEM((2, CR, 128), jnp.int32),
            pltpu.VMEM((2, CR, 128), jnp.float32),
            pltpu.VMEM_SHARED((NPAD * G,), jnp.float32),
            pltpu.VMEM_SHARED((NPAD,), jnp.float32),
            pltpu.VMEM_SHARED((NPAD,), jnp.int32),
            pltpu.VMEM_SHARED((G,), jnp.float32),
            pltpu.SemaphoreType.DMA,
            pltpu.SemaphoreType.DMA,
            pltpu.SemaphoreType.DMA,
            pltpu.SemaphoreType.DMA,
        ],
    )
    return f(srce2d, dste2d, batch_pad)


def _tc_body(d_ref, dinv_ref, cnt_ref, x_ref, w1_ref, b1_ref, wlin_ref,
             blin_ref, out_ref):
    cs = d_ref[:N, :] * dinv_ref[:N][:, None]
    a = lax.dot_general(cs, x_ref[...], (((0,), (0,)), ((), ())),
                        preferred_element_type=jnp.float32)
    cnt = cnt_ref[...]
    h = jnp.dot(a, w1_ref[...], preferred_element_type=jnp.float32)
    h = h + cnt[:, None] * b1_ref[...][None, :]
    pooled = h / jnp.maximum(cnt, 1.0)[:, None]
    out_ref[...] = (jnp.dot(pooled, wlin_ref[...],
                            preferred_element_type=jnp.float32)
                    + blin_ref[...][None, :])


@jax.jit
def _tc_dense(d_flat, dinv, cnt, x, W1, b1, Wlin, blin):
    d2 = d_flat.reshape(NPAD, G)
    return pl.pallas_call(
        _tc_body,
        out_shape=jax.ShapeDtypeStruct((G, OUT), jnp.float32),
    )(d2, dinv, cnt, x, W1, b1, Wlin, blin)


@jax.jit
def _prep(edge_index, batch):
    pad = jnp.full((EROWS * 128 - E,), N, jnp.int32)
    srce2d = jnp.concatenate([edge_index[0], pad]).reshape(EROWS, 128)
    dste2d = jnp.concatenate([edge_index[1], pad]).reshape(EROWS, 128)
    batch_pad = jnp.concatenate(
        [batch, jnp.zeros((NPAD - N,), jnp.int32)])
    return srce2d, dste2d, batch_pad


def kernel(x, edge_index, batch, W1, b1, Wlin, blin):
    srce2d, dste2d, batch_pad = _prep(edge_index, batch)
    d_flat, dinv, cnt = _sc_coeffs(srce2d, dste2d, batch_pad)
    return _tc_dense(d_flat, dinv, cnt, x, W1, b1, Wlin, blin)


# speedup vs baseline: 1.0204x; 1.0204x over previous
import jax
import jax.numpy as jnp
from jax import lax
from jax.experimental import pallas as pl
from jax.experimental.pallas import tpu as pltpu
from jax.experimental.pallas import tpu_sc as plsc

N = 10000
E = 320000
F_IN = 128
HID = 128
OUT = 128
G = 128

NS = 16
L = 16

NPAD = 10240
NT = NPAD // NS
RPT = 160
EROWS = NS * RPT
CSL = NPAD * G // NS
ZB = 2048
CR = 16
CHUNKS = [(q * CR, CR) for q in range(RPT // CR)]


def _rsqrt16(x):
    i = plsc.bitcast(x, jnp.int32)
    y = plsc.bitcast(jnp.int32(0x5F3759DF) - (i >> 1), jnp.float32)
    for _ in range(3):
        y = y * (1.5 - 0.5 * x * y * y)
    return y


def _sc_body(srce_hbm, dste_hbm, batch_hbm, d_out, dinv_out, cnt_out,
             combo_v, src_c, dst_c, acc_s, bat_s, zbuf_v,
             idx_c, val_c,
             D_sh, deg_sh, combo_sh, cnt_sh,
             sem_stage, sem_scat, sem_zero, sem_bat):
    sid = lax.axis_index("s")

    zero16f = jnp.zeros((L,), jnp.float32)
    one16f = jnp.ones((L,), jnp.float32)
    iota16 = lax.broadcasted_iota(jnp.int32, (L,), 0)
    rbase = sid * RPT
    sb = sid * NT

    stage_bat = pltpu.async_copy(batch_hbm.at[pl.ds(sb, NT)], bat_s,
                                 sem_bat)

    def zz(i, c):
        zbuf_v[pl.ds(i * L, L)] = zero16f
        return c
    lax.fori_loop(0, ZB // L, zz, 0)
    zdeg_d = pltpu.async_copy(zbuf_v.at[pl.ds(0, NT)],
                              deg_sh.at[pl.ds(sb, NT)], sem_zero)
    zdescs = [pltpu.async_copy(zbuf_v,
                               D_sh.at[pl.ds(sid * CSL + q * ZB, ZB)],
                               sem_zero)
              for q in range(CSL // ZB)]
    zdescs.append(pltpu.async_copy(zbuf_v.at[pl.ds(0, G)], cnt_sh,
                                   sem_zero))

    for p in range(2):
        def fill1(i, c, p=p):
            val_c[p, i // 8, pl.ds((i % 8) * L, L)] = one16f
            return c
        lax.fori_loop(0, (CR * 128) // L, fill1, 0)
    zdeg_d.wait()
    plsc.subcore_barrier()

    stage_d = {0: pltpu.async_copy(
        dste_hbm.at[pl.ds(rbase, CR)], idx_c.at[0], sem_stage)}
    scat_d = {}
    for k, (off, rk) in enumerate(CHUNKS):
        p = k % 2
        stage_d.pop(k).wait()
        scat_d[k] = [pltpu.async_copy(val_c.at[p, j],
                                      deg_sh.at[idx_c.at[p, j]],
                                      sem_scat, add=True)
                     for j in range(rk)]
        if k - 1 in scat_d:
            for d in scat_d.pop(k - 1):
                d.wait()
        if k + 1 < len(CHUNKS):
            stage_d[k + 1] = pltpu.async_copy(
                dste_hbm.at[pl.ds(rbase + CHUNKS[k + 1][0], CR)],
                idx_c.at[1 - p], sem_stage)
    for d in scat_d.pop(len(CHUNKS) - 1):
        d.wait()

    plsc.subcore_barrier()

    pltpu.sync_copy(deg_sh.at[pl.ds(sb, NT)], acc_s)
    stage_bat.wait()

    def dinv_calc(q, c):
        dv = _rsqrt16(acc_s[pl.ds(q * L, L)] + 1.0)
        acc_s[pl.ds(q * L, L)] = dv
        combo_v[pl.ds(q * L, L)] = (
            (plsc.bitcast(dv, jnp.int32) & jnp.int32(-128))
            | bat_s[pl.ds(q * L, L)])
        return c
    lax.fori_loop(0, NT // L, dinv_calc, 0)
    pltpu.sync_copy(combo_v.at[pl.ds(0, NT)], combo_sh.at[pl.ds(sb, NT)])
    pltpu.sync_copy(acc_s, dinv_out.at[pl.ds(sb, NT)])

    for d in zdescs:
        d.wait()
    plsc.subcore_barrier()
    combo_d = pltpu.async_copy(combo_sh, combo_v, sem_bat)

    stage_d = {0: [pltpu.async_copy(
        srce_hbm.at[pl.ds(rbase, CR)], src_c.at[0], sem_stage),
        pltpu.async_copy(
        dste_hbm.at[pl.ds(rbase, CR)], dst_c.at[0], sem_stage)]}
    scat_d = {}
    for k, (off, rk) in enumerate(CHUNKS):
        p = k % 2
        if k == 0:
            combo_d.wait()
        for d in stage_d.pop(k):
            d.wait()
        if k - 2 in scat_d:
            for d in scat_d.pop(k - 2):
                d.wait()

        def coeff(i, c, p=p):
            r = i // 8
            cl = (i % 8) * L
            s16 = src_c[p, r, pl.ds(cl, L)]
            d16 = dst_c[p, r, pl.ds(cl, L)]
            cb = plsc.load_gather(combo_v, [d16])
            idx_c[p, r, pl.ds(cl, L)] = s16 * G + (cb & jnp.int32(127))
            val_c[p, r, pl.ds(cl, L)] = plsc.bitcast(
                cb & jnp.int32(-128), jnp.float32)
            return c
        lax.fori_loop(0, (rk * 128) // L, coeff, 0)
        scat_d[k] = [pltpu.async_copy(val_c.at[p, j],
                                      D_sh.at[idx_c.at[p, j]],
                                      sem_scat, add=True)
                     for j in range(rk)]
        if k + 1 < len(CHUNKS):
            off2 = CHUNKS[k + 1][0]
            stage_d[k + 1] = [pltpu.async_copy(
                srce_hbm.at[pl.ds(rbase + off2, CR)],
                src_c.at[1 - p], sem_stage),
                pltpu.async_copy(
                dste_hbm.at[pl.ds(rbase + off2, CR)],
                dst_c.at[1 - p], sem_stage)]
    for k in list(scat_d):
        for d in scat_d.pop(k):
            d.wait()

    for q in range(NT // L):
        k = q * L
        nvec = sb + k + iota16
        dv = acc_s[pl.ds(k, L)]
        g16 = bat_s[pl.ds(k, L)]
        valid = nvec < N
        idx_c[0, k // 128, pl.ds(k % 128, L)] = jnp.where(
            valid, nvec * G + g16, 0)
        val_c[0, k // 128, pl.ds(k % 128, L)] = jnp.where(valid, dv, 0.0)
    sl_d = [pltpu.async_copy(val_c.at[0, j], D_sh.at[idx_c.at[0, j]],
                             sem_scat, add=True)
            for j in range(NT // 128)]

    for q in range(NT // L):
        k = q * L
        nvec = sb + k + iota16
        valid = nvec < N
        idx_c[1, k // 128, pl.ds(k % 128, L)] = jnp.where(
            valid, bat_s[pl.ds(k, L)], 0)
        val_c[1, k // 128, pl.ds(k % 128, L)] = jnp.where(valid, 1.0, 0.0)
    sl_d += [pltpu.async_copy(val_c.at[1, j], cnt_sh.at[idx_c.at[1, j]],
                              sem_scat, add=True)
             for j in range(NT // 128)]
    for d in sl_d:
        d.wait()

    plsc.subcore_barrier()

    pltpu.sync_copy(D_sh.at[pl.ds(sid * CSL, CSL)],
                    d_out.at[pl.ds(sid * CSL, CSL)])

    @pl.when(sid == 0)
    def _():
        pltpu.sync_copy(cnt_sh, cnt_out)


@jax.jit
def _sc_coeffs(srce2d, dste2d, batch_pad):
    mesh = plsc.VectorSubcoreMesh(core_axis_name="c", subcore_axis_name="s",
                                  num_cores=1, num_subcores=NS)
    f = pl.kernel(
        _sc_body,
        out_type=(jax.ShapeDtypeStruct((NPAD * G,), jnp.float32),
                  jax.ShapeDtypeStruct((NPAD,), jnp.float32),
                  jax.ShapeDtypeStruct((G,), jnp.float32)),
        mesh=mesh,
        compiler_params=pltpu.CompilerParams(needs_layout_passes=False),
        scratch_types=[
            pltpu.VMEM((NPAD,), jnp.int32),
            pltpu.VMEM((2, CR, 128), jnp.int32),
            pltpu.VMEM((2, CR, 128), jnp.int32),
            pltpu.VMEM((NT,), jnp.float32),
            pltpu.VMEM((NT,), jnp.int32),
            pltpu.VMEM((ZB,), jnp.float32),
            pltpu.VMEM((2, CR, 128), jnp.int32),
            pltpu.VMEM((2, CR, 128), jnp.float32),
            pltpu.VMEM_SHARED((NPAD * G,), jnp.float32),
            pltpu.VMEM_SHARED((NPAD,), jnp.float32),
            pltpu.VMEM_SHARED((NPAD,), jnp.int32),
            pltpu.VMEM_SHARED((G,), jnp.float32),
            pltpu.SemaphoreType.DMA,
            pltpu.SemaphoreType.DMA,
            pltpu.SemaphoreType.DMA,
            pltpu.SemaphoreType.DMA,
        ],
    )
    return f(srce2d, dste2d, batch_pad)


def _tc_body(d_ref, dinv_ref, cnt_ref, x_ref, w1_ref, b1_ref, wlin_ref,
             blin_ref, out_ref):
    cs = d_ref[:N, :] * dinv_ref[:N][:, None]
    a = lax.dot_general(cs, x_ref[...], (((0,), (0,)), ((), ())),
                        preferred_element_type=jnp.float32)
    cnt = cnt_ref[...]
    h = jnp.dot(a, w1_ref[...], preferred_element_type=jnp.float32)
    h = h + cnt[:, None] * b1_ref[...][None, :]
    pooled = h / jnp.maximum(cnt, 1.0)[:, None]
    out_ref[...] = (jnp.dot(pooled, wlin_ref[...],
                            preferred_element_type=jnp.float32)
                    + blin_ref[...][None, :])


@jax.jit
def _tc_dense(d_flat, dinv, cnt, x, W1, b1, Wlin, blin):
    d2 = d_flat.reshape(NPAD, G)
    return pl.pallas_call(
        _tc_body,
        out_shape=jax.ShapeDtypeStruct((G, OUT), jnp.float32),
    )(d2, dinv, cnt, x, W1, b1, Wlin, blin)


@jax.jit
def _prep(edge_index, batch):
    pad = jnp.full((EROWS * 128 - E,), N, jnp.int32)
    srce2d = jnp.concatenate([edge_index[0], pad]).reshape(EROWS, 128)
    dste2d = jnp.concatenate([edge_index[1], pad]).reshape(EROWS, 128)
    batch_pad = jnp.concatenate(
        [batch, jnp.zeros((NPAD - N,), jnp.int32)])
    return srce2d, dste2d, batch_pad


def kernel(x, edge_index, batch, W1, b1, Wlin, blin):
    srce2d, dste2d, batch_pad = _prep(edge_index, batch)
    d_flat, dinv, cnt = _sc_coeffs(srce2d, dste2d, batch_pad)
    return _tc_dense(d_flat, dinv, cnt, x, W1, b1, Wlin, blin)


# speedup vs baseline: 1.1565x; 1.1334x over previous
import jax
import jax.numpy as jnp
from jax import lax
from jax.experimental import pallas as pl
from jax.experimental.pallas import tpu as pltpu
from jax.experimental.pallas import tpu_sc as plsc

N = 10000
E = 320000
F_IN = 128
HID = 128
OUT = 128
G = 128

NS = 16
L = 16

NPAD = 10240
NT = NPAD // NS
RPT = 160
EROWS = NS * RPT
CSL = NPAD * G // NS
ZB = 2048
CR = 16
CHUNKS = [(q * CR, CR) for q in range(RPT // CR)]


def _rsqrt16(x):
    i = plsc.bitcast(x, jnp.int32)
    y = plsc.bitcast(jnp.int32(0x5F3759DF) - (i >> 1), jnp.float32)
    for _ in range(3):
        y = y * (1.5 - 0.5 * x * y * y)
    return y


def _sc_body(ei_hbm, batch_hbm, d_out, dinv_out, cnt_out,
             combo_v, src_c, dst_c, acc_s, bat_s, zbuf_v,
             idx_c, val_c,
             D_sh, deg_sh, combo_sh, cnt_sh,
             sem_stage, sem_scat, sem_zero, sem_bat):
    sid = lax.axis_index("s")

    zero16f = jnp.zeros((L,), jnp.float32)
    one16f = jnp.ones((L,), jnp.float32)
    iota16 = lax.broadcasted_iota(jnp.int32, (L,), 0)
    rbase = sid * RPT
    sb = sid * NT

    stage_bat = pltpu.async_copy(batch_hbm.at[pl.ds(sb, NT)], bat_s,
                                 sem_bat)

    def zz(i, c):
        zbuf_v[pl.ds(i * L, L)] = zero16f
        return c
    lax.fori_loop(0, ZB // L, zz, 0)
    zdeg_d = pltpu.async_copy(zbuf_v.at[pl.ds(0, NT)],
                              deg_sh.at[pl.ds(sb, NT)], sem_zero)
    zdescs = [pltpu.async_copy(zbuf_v,
                               D_sh.at[pl.ds(sid * CSL + q * ZB, ZB)],
                               sem_zero)
              for q in range(CSL // ZB)]
    zdescs.append(pltpu.async_copy(zbuf_v.at[pl.ds(0, G)], cnt_sh,
                                   sem_zero))

    for p in range(2):
        def fill1(i, c, p=p):
            val_c[p, i // 8, pl.ds((i % 8) * L, L)] = one16f
            return c
        lax.fori_loop(0, (CR * 128) // L, fill1, 0)
    zdeg_d.wait()
    plsc.subcore_barrier()

    stage_d = {0: pltpu.async_copy(
        ei_hbm.at[1, pl.ds(rbase, CR)], idx_c.at[0], sem_stage)}
    scat_d = {}
    for k, (off, rk) in enumerate(CHUNKS):
        p = k % 2
        stage_d.pop(k).wait()
        scat_d[k] = [pltpu.async_copy(val_c.at[p, j],
                                      deg_sh.at[idx_c.at[p, j]],
                                      sem_scat, add=True)
                     for j in range(rk)]
        if k - 1 in scat_d:
            for d in scat_d.pop(k - 1):
                d.wait()
        if k + 1 < len(CHUNKS):
            stage_d[k + 1] = pltpu.async_copy(
                ei_hbm.at[1, pl.ds(rbase + CHUNKS[k + 1][0], CR)],
                idx_c.at[1 - p], sem_stage)
    for d in scat_d.pop(len(CHUNKS) - 1):
        d.wait()

    plsc.subcore_barrier()

    pltpu.sync_copy(deg_sh.at[pl.ds(sb, NT)], acc_s)
    stage_bat.wait()

    def dinv_calc(q, c):
        dv = _rsqrt16(acc_s[pl.ds(q * L, L)] + 1.0)
        acc_s[pl.ds(q * L, L)] = dv
        combo_v[pl.ds(q * L, L)] = (
            (plsc.bitcast(dv, jnp.int32) & jnp.int32(-128))
            | bat_s[pl.ds(q * L, L)])
        return c
    lax.fori_loop(0, NT // L, dinv_calc, 0)
    pltpu.sync_copy(combo_v.at[pl.ds(0, NT)], combo_sh.at[pl.ds(sb, NT)])
    pltpu.sync_copy(acc_s, dinv_out.at[pl.ds(sb, NT)])

    for d in zdescs:
        d.wait()
    plsc.subcore_barrier()
    combo_d = pltpu.async_copy(combo_sh, combo_v, sem_bat)

    stage_d = {0: [pltpu.async_copy(
        ei_hbm.at[0, pl.ds(rbase, CR)], src_c.at[0], sem_stage),
        pltpu.async_copy(
        ei_hbm.at[1, pl.ds(rbase, CR)], dst_c.at[0], sem_stage)]}
    scat_d = {}
    for k, (off, rk) in enumerate(CHUNKS):
        p = k % 2
        if k == 0:
            combo_d.wait()
        for d in stage_d.pop(k):
            d.wait()
        if k - 2 in scat_d:
            for d in scat_d.pop(k - 2):
                d.wait()

        def coeff(i, c, p=p):
            r = i // 8
            cl = (i % 8) * L
            s16 = src_c[p, r, pl.ds(cl, L)]
            d16 = dst_c[p, r, pl.ds(cl, L)]
            cb = plsc.load_gather(combo_v, [d16])
            idx_c[p, r, pl.ds(cl, L)] = s16 * G + (cb & jnp.int32(127))
            val_c[p, r, pl.ds(cl, L)] = plsc.bitcast(
                cb & jnp.int32(-128), jnp.float32)
            return c
        lax.fori_loop(0, (rk * 128) // L, coeff, 0)
        scat_d[k] = [pltpu.async_copy(val_c.at[p, j],
                                      D_sh.at[idx_c.at[p, j]],
                                      sem_scat, add=True)
                     for j in range(rk)]
        if k + 1 < len(CHUNKS):
            off2 = CHUNKS[k + 1][0]
            stage_d[k + 1] = [pltpu.async_copy(
                ei_hbm.at[0, pl.ds(rbase + off2, CR)],
                src_c.at[1 - p], sem_stage),
                pltpu.async_copy(
                ei_hbm.at[1, pl.ds(rbase + off2, CR)],
                dst_c.at[1 - p], sem_stage)]
    for k in list(scat_d):
        for d in scat_d.pop(k):
            d.wait()

    for q in range(NT // L):
        k = q * L
        nvec = sb + k + iota16
        dv = acc_s[pl.ds(k, L)]
        g16 = bat_s[pl.ds(k, L)]
        valid = nvec < N
        idx_c[0, k // 128, pl.ds(k % 128, L)] = jnp.where(
            valid, nvec * G + g16, 0)
        val_c[0, k // 128, pl.ds(k % 128, L)] = jnp.where(valid, dv, 0.0)
    sl_d = [pltpu.async_copy(val_c.at[0, j], D_sh.at[idx_c.at[0, j]],
                             sem_scat, add=True)
            for j in range(NT // 128)]

    for q in range(NT // L):
        k = q * L
        nvec = sb + k + iota16
        valid = nvec < N
        idx_c[1, k // 128, pl.ds(k % 128, L)] = jnp.where(
            valid, bat_s[pl.ds(k, L)], 0)
        val_c[1, k // 128, pl.ds(k % 128, L)] = jnp.where(valid, 1.0, 0.0)
    sl_d += [pltpu.async_copy(val_c.at[1, j], cnt_sh.at[idx_c.at[1, j]],
                              sem_scat, add=True)
             for j in range(NT // 128)]
    for d in sl_d:
        d.wait()

    plsc.subcore_barrier()

    pltpu.sync_copy(D_sh.at[pl.ds(sid * CSL, CSL)],
                    d_out.at[pl.ds(sid * CSL, CSL)])

    @pl.when(sid == 0)
    def _():
        pltpu.sync_copy(cnt_sh, cnt_out)


@jax.jit
def _sc_coeffs(ei3, batch_pad):
    mesh = plsc.VectorSubcoreMesh(core_axis_name="c", subcore_axis_name="s",
                                  num_cores=1, num_subcores=NS)
    f = pl.kernel(
        _sc_body,
        out_type=(jax.ShapeDtypeStruct((NPAD * G,), jnp.float32),
                  jax.ShapeDtypeStruct((NPAD,), jnp.float32),
                  jax.ShapeDtypeStruct((G,), jnp.float32)),
        mesh=mesh,
        compiler_params=pltpu.CompilerParams(needs_layout_passes=False),
        scratch_types=[
            pltpu.VMEM((NPAD,), jnp.int32),
            pltpu.VMEM((2, CR, 128), jnp.int32),
            pltpu.VMEM((2, CR, 128), jnp.int32),
            pltpu.VMEM((NT,), jnp.float32),
            pltpu.VMEM((NT,), jnp.int32),
            pltpu.VMEM((ZB,), jnp.float32),
            pltpu.VMEM((2, CR, 128), jnp.int32),
            pltpu.VMEM((2, CR, 128), jnp.float32),
            pltpu.VMEM_SHARED((NPAD * G,), jnp.float32),
            pltpu.VMEM_SHARED((NPAD,), jnp.float32),
            pltpu.VMEM_SHARED((NPAD,), jnp.int32),
            pltpu.VMEM_SHARED((G,), jnp.float32),
            pltpu.SemaphoreType.DMA,
            pltpu.SemaphoreType.DMA,
            pltpu.SemaphoreType.DMA,
            pltpu.SemaphoreType.DMA,
        ],
    )
    return f(ei3, batch_pad)


def _tc_body(d_ref, dinv_ref, cnt_ref, x_ref, w1_ref, b1_ref, wlin_ref,
             blin_ref, out_ref):
    cs = d_ref[:N, :] * dinv_ref[:N][:, None]
    a = lax.dot_general(cs, x_ref[...], (((0,), (0,)), ((), ())),
                        preferred_element_type=jnp.float32)
    cnt = cnt_ref[...]
    h = jnp.dot(a, w1_ref[...], preferred_element_type=jnp.float32)
    h = h + cnt[:, None] * b1_ref[...][None, :]
    pooled = h / jnp.maximum(cnt, 1.0)[:, None]
    out_ref[...] = (jnp.dot(pooled, wlin_ref[...],
                            preferred_element_type=jnp.float32)
                    + blin_ref[...][None, :])


@jax.jit
def _tc_dense(d_flat, dinv, cnt, x, W1, b1, Wlin, blin):
    d2 = d_flat.reshape(NPAD, G)
    return pl.pallas_call(
        _tc_body,
        out_shape=jax.ShapeDtypeStruct((G, OUT), jnp.float32),
    )(d2, dinv, cnt, x, W1, b1, Wlin, blin)


@jax.jit
def _prep(edge_index, batch):
    pad = jnp.full((2, EROWS * 128 - E), N, jnp.int32)
    ei3 = jnp.concatenate([edge_index, pad], axis=1).reshape(2, EROWS, 128)
    batch_pad = jnp.concatenate(
        [batch, jnp.zeros((NPAD - N,), jnp.int32)])
    return ei3, batch_pad


def kernel(x, edge_index, batch, W1, b1, Wlin, blin):
    ei3, batch_pad = _prep(edge_index, batch)
    d_flat, dinv, cnt = _sc_coeffs(ei3, batch_pad)
    return _tc_dense(d_flat, dinv, cnt, x, W1, b1, Wlin, blin)
